# baseline (speedup 1.0000x reference)
import functools

import jax
import jax.numpy as jnp
from jax import lax
from jax.experimental import pallas as pl
from jax.experimental.pallas import tpu as pltpu
from jax.experimental.pallas import tpu_sc as plsc

_PACK = 4


def _repack_body(t_ref, o_ref):
    D = t_ref.shape[1]
    for j in range(_PACK):
        o_ref[:, j * D:(j + 1) * D] = t_ref[j::_PACK, :]


def _repack(table, rt=4000):
    V, D = table.shape
    grid = (V // rt,)
    return pl.pallas_call(
        _repack_body,
        grid=grid,
        in_specs=[pl.BlockSpec((rt, D), lambda i: (i, 0))],
        out_specs=pl.BlockSpec((rt // _PACK, D * _PACK), lambda i: (i, 0)),
        out_shape=jax.ShapeDtypeStruct((V // _PACK, D * _PACK), jnp.float32),
    )(table)


@functools.lru_cache(maxsize=None)
def _make_sc_gather(NSLAB, W, B):
    info = plsc.get_sparse_core_info()
    NC, NSUB = info.num_cores, info.num_subcores
    NW = NC * NSUB
    assert B % (8 * NW) == 0
    b_per_w = B // NW
    mesh = plsc.VectorSubcoreMesh(core_axis_name="c", subcore_axis_name="s")

    @functools.partial(
        pl.kernel,
        mesh=mesh,
        out_type=jax.ShapeDtypeStruct((B, W), jnp.float32),
        scratch_types=[
            pltpu.VMEM((b_per_w,), jnp.int32),
            pltpu.VMEM((b_per_w, W), jnp.float32),
            pltpu.SemaphoreType.DMA,
        ],
    )
    def gather(packed_hbm, idx_hbm, out_hbm, idx_v, slabs_v, sem):
        wid = lax.axis_index("s") * NC + lax.axis_index("c")
        base = wid * b_per_w
        pltpu.sync_copy(idx_hbm.at[pl.ds(base, b_per_w)], idx_v)
        pltpu.async_copy(packed_hbm.at[idx_v], slabs_v, sem).wait()
        pltpu.sync_copy(slabs_v, out_hbm.at[pl.ds(base, b_per_w)])

    return gather


def _proj_body(x4_ref, oh_ref, w_ref, b_ref, o_ref, x_ref):
    @pl.when(pl.program_id(0) == 0)
    def _():
        D = x_ref.shape[1]
        acc = x4_ref[:, 0:D] * oh_ref[:, 0:1]
        for j in range(1, _PACK):
            acc += x4_ref[:, j * D:(j + 1) * D] * oh_ref[:, j:j + 1]
        x_ref[...] = acc

    o_ref[...] = (lax.dot_general(
        x_ref[...], w_ref[...],
        (((1,), (1,)), ((), ())),
        preferred_element_type=jnp.float32,
    ) + b_ref[...])[:, None, :]


def _project(x4, onehot, proj_w, proj_b, vt=2048):
    B = x4.shape[0]
    V, D = proj_w.shape
    grid = (pl.cdiv(V, vt),)
    return pl.pallas_call(
        _proj_body,
        grid=grid,
        in_specs=[
            pl.BlockSpec((B, D * _PACK), lambda i: (0, 0)),
            pl.BlockSpec((B, _PACK), lambda i: (0, 0)),
            pl.BlockSpec((vt, D), lambda i: (i, 0)),
            pl.BlockSpec((1, vt), lambda i: (0, i)),
        ],
        out_specs=pl.BlockSpec((B, 1, vt), lambda i: (0, 0, i)),
        out_shape=jax.ShapeDtypeStruct((B, 1, V), jnp.float32),
        scratch_shapes=[pltpu.VMEM((B, D), jnp.float32)],
    )(x4, onehot, proj_w, proj_b.reshape(1, V))


def kernel(input_tokens, token_embeddings, proj_w, proj_b):
    tokens = input_tokens.reshape(-1).astype(jnp.int32)
    B = tokens.shape[0]
    V, D = token_embeddings.shape
    packed = _repack(token_embeddings)
    slab_idx = tokens // _PACK
    onehot = (tokens[:, None] % _PACK ==
              jnp.arange(_PACK, dtype=jnp.int32)).astype(jnp.float32)
    x4 = _make_sc_gather(V // _PACK, D * _PACK, B)(packed, slab_idx)
    return _project(x4, onehot, proj_w, proj_b)


# speedup vs baseline: 6.3012x; 6.3012x over previous
import functools

import jax
import jax.numpy as jnp
from jax import lax
from jax.experimental import pallas as pl
from jax.experimental.pallas import tpu as pltpu
from jax.experimental.pallas import tpu_sc as plsc

_PACK = 4


def _repack_body(tT_ref, o_ref, s_ref):
    s_ref[...] = tT_ref[...].T
    D = tT_ref.shape[0]
    for j in range(_PACK):
        o_ref[:, j * D:(j + 1) * D] = s_ref[j::_PACK, :]


def _repack(tableT, rt=4096):
    D, V = tableT.shape
    grid = (pl.cdiv(V, rt),)
    return pl.pallas_call(
        _repack_body,
        grid=grid,
        in_specs=[pl.BlockSpec((D, rt), lambda i: (0, i))],
        out_specs=pl.BlockSpec((rt // _PACK, D * _PACK), lambda i: (i, 0)),
        out_shape=jax.ShapeDtypeStruct((V // _PACK, D * _PACK), jnp.float32),
        scratch_shapes=[pltpu.VMEM((rt, D), jnp.float32)],
    )(tableT)


@functools.lru_cache(maxsize=None)
def _make_sc_gather(NSLAB, W, B):
    info = plsc.get_sparse_core_info()
    NC, NSUB = info.num_cores, info.num_subcores
    NW = NC * NSUB
    assert B % (8 * NW) == 0
    b_per_w = B // NW
    mesh = plsc.VectorSubcoreMesh(core_axis_name="c", subcore_axis_name="s")

    @functools.partial(
        pl.kernel,
        mesh=mesh,
        out_type=jax.ShapeDtypeStruct((B, W), jnp.float32),
        scratch_types=[
            pltpu.VMEM((b_per_w,), jnp.int32),
            pltpu.VMEM((b_per_w, W), jnp.float32),
            pltpu.SemaphoreType.DMA,
        ],
    )
    def gather(packed_hbm, idx_hbm, out_hbm, idx_v, slabs_v, sem):
        wid = lax.axis_index("s") * NC + lax.axis_index("c")
        base = wid * b_per_w
        pltpu.sync_copy(idx_hbm.at[pl.ds(base, b_per_w)], idx_v)
        pltpu.async_copy(packed_hbm.at[idx_v], slabs_v, sem).wait()
        pltpu.sync_copy(slabs_v, out_hbm.at[pl.ds(base, b_per_w)])

    return gather


def _proj_body(x4_ref, oh_ref, wT_ref, b_ref, o_ref, x_ref):
    @pl.when(pl.program_id(0) == 0)
    def _():
        D = x_ref.shape[1]
        acc = x4_ref[:, 0:D] * oh_ref[:, 0:1]
        for j in range(1, _PACK):
            acc += x4_ref[:, j * D:(j + 1) * D] * oh_ref[:, j:j + 1]
        x_ref[...] = acc

    B = x_ref.shape[0]
    o_ref[...] = lax.dot_general(
        wT_ref[...], x_ref[...],
        (((0,), (1,)), ((), ())),
        preferred_element_type=jnp.float32,
    ) + lax.dot_general(
        b_ref[...], jnp.ones((1, B), jnp.float32),
        (((0,), (0,)), ((), ())),
        preferred_element_type=jnp.float32,
    )


def _project(x4, onehot, proj_wT, proj_b2, vt=2048):
    B = x4.shape[0]
    D, V = proj_wT.shape
    grid = (pl.cdiv(V, vt),)
    return pl.pallas_call(
        _proj_body,
        grid=grid,
        in_specs=[
            pl.BlockSpec((B, D * _PACK), lambda i: (0, 0)),
            pl.BlockSpec((B, _PACK), lambda i: (0, 0)),
            pl.BlockSpec((D, vt), lambda i: (0, i)),
            pl.BlockSpec((1, vt), lambda i: (0, i)),
        ],
        out_specs=pl.BlockSpec((vt, B), lambda i: (i, 0)),
        out_shape=jax.ShapeDtypeStruct((V, B), jnp.float32),
        scratch_shapes=[pltpu.VMEM((B, D), jnp.float32)],
    )(x4, onehot, proj_wT, proj_b2)


def kernel(input_tokens, token_embeddings, proj_w, proj_b):
    tokens = input_tokens.reshape(-1).astype(jnp.int32)
    B = tokens.shape[0]
    V, D = token_embeddings.shape
    packed = _repack(jnp.swapaxes(token_embeddings, 0, 1))
    slab_idx = tokens // _PACK
    onehot = (tokens[:, None] % _PACK ==
              jnp.arange(_PACK, dtype=jnp.int32)).astype(jnp.float32)
    x4 = _make_sc_gather(V // _PACK, D * _PACK, B)(packed, slab_idx)
    logits_t = _project(x4, onehot, jnp.swapaxes(proj_w, 0, 1),
                        proj_b.reshape(1, V))
    return jnp.swapaxes(logits_t, 0, 1)[:, None, :]


# speedup vs baseline: 6.3018x; 1.0001x over previous
import functools

import jax
import jax.numpy as jnp
from jax import lax
from jax.experimental import pallas as pl
from jax.experimental.pallas import tpu as pltpu
from jax.experimental.pallas import tpu_sc as plsc

_PACK = 4


def _repack_body(tT_ref, o_ref, s_ref):
    s_ref[...] = tT_ref[...].T
    D = tT_ref.shape[0]
    for j in range(_PACK):
        o_ref[:, j * D:(j + 1) * D] = s_ref[j::_PACK, :]


def _repack(tableT, rt=4096):
    D, V = tableT.shape
    grid = (pl.cdiv(V, rt),)
    return pl.pallas_call(
        _repack_body,
        grid=grid,
        in_specs=[pl.BlockSpec((D, rt), lambda i: (0, i))],
        out_specs=pl.BlockSpec((rt // _PACK, D * _PACK), lambda i: (i, 0)),
        out_shape=jax.ShapeDtypeStruct((V // _PACK, D * _PACK), jnp.float32),
        scratch_shapes=[pltpu.VMEM((rt, D), jnp.float32)],
    )(tableT)


@functools.lru_cache(maxsize=None)
def _make_sc_gather(NSLAB, W, B):
    info = plsc.get_sparse_core_info()
    NC, NSUB = info.num_cores, info.num_subcores
    NW = NC * NSUB
    assert B % (8 * NW) == 0
    b_per_w = B // NW
    mesh = plsc.VectorSubcoreMesh(core_axis_name="c", subcore_axis_name="s")

    @functools.partial(
        pl.kernel,
        mesh=mesh,
        out_type=jax.ShapeDtypeStruct((B, W), jnp.float32),
        scratch_types=[
            pltpu.VMEM((b_per_w,), jnp.int32),
            pltpu.VMEM((b_per_w, W), jnp.float32),
            pltpu.SemaphoreType.DMA,
        ],
    )
    def gather(packed_hbm, idx_hbm, out_hbm, idx_v, slabs_v, sem):
        wid = lax.axis_index("s") * NC + lax.axis_index("c")
        base = wid * b_per_w
        pltpu.sync_copy(idx_hbm.at[pl.ds(base, b_per_w)], idx_v)
        pltpu.async_copy(packed_hbm.at[idx_v], slabs_v, sem).wait()
        pltpu.sync_copy(slabs_v, out_hbm.at[pl.ds(base, b_per_w)])

    return gather


def _proj_body(x4_ref, oh_ref, wT_ref, b_ref, o_ref, x_ref):
    @pl.when(pl.program_id(0) == 0)
    def _():
        D = x_ref.shape[1]
        acc = x4_ref[:, 0:D] * oh_ref[:, 0:1]
        for j in range(1, _PACK):
            acc += x4_ref[:, j * D:(j + 1) * D] * oh_ref[:, j:j + 1]
        x_ref[...] = acc

    B = x_ref.shape[0]
    o_ref[...] = lax.dot_general(
        wT_ref[...], x_ref[...],
        (((0,), (1,)), ((), ())),
        preferred_element_type=jnp.float32,
    ) + lax.dot_general(
        b_ref[...], jnp.ones((1, B), jnp.float32),
        (((0,), (0,)), ((), ())),
        preferred_element_type=jnp.float32,
    )


def _project(x4, onehot, proj_wT, proj_b2, vt=4096):
    B = x4.shape[0]
    D, V = proj_wT.shape
    grid = (pl.cdiv(V, vt),)
    return pl.pallas_call(
        _proj_body,
        grid=grid,
        in_specs=[
            pl.BlockSpec((B, D * _PACK), lambda i: (0, 0)),
            pl.BlockSpec((B, _PACK), lambda i: (0, 0)),
            pl.BlockSpec((D, vt), lambda i: (0, i)),
            pl.BlockSpec((1, vt), lambda i: (0, i)),
        ],
        out_specs=pl.BlockSpec((vt, B), lambda i: (i, 0)),
        out_shape=jax.ShapeDtypeStruct((V, B), jnp.float32),
        scratch_shapes=[pltpu.VMEM((B, D), jnp.float32)],
    )(x4, onehot, proj_wT, proj_b2)


def kernel(input_tokens, token_embeddings, proj_w, proj_b):
    tokens = input_tokens.reshape(-1).astype(jnp.int32)
    B = tokens.shape[0]
    V, D = token_embeddings.shape
    packed = _repack(jnp.swapaxes(token_embeddings, 0, 1))
    slab_idx = tokens // _PACK
    onehot = (tokens[:, None] % _PACK ==
              jnp.arange(_PACK, dtype=jnp.int32)).astype(jnp.float32)
    x4 = _make_sc_gather(V // _PACK, D * _PACK, B)(packed, slab_idx)
    logits_t = _project(x4, onehot, jnp.swapaxes(proj_w, 0, 1),
                        proj_b.reshape(1, V))
    return jnp.swapaxes(logits_t, 0, 1)[:, None, :]
